# Initial kernel scaffold; baseline (speedup 1.0000x reference)
import jax
import jax.numpy as jnp
from jax.experimental import pallas as pl


def kernel(in_feat, edge_index, W1_self, W1_neigh, b1, W2_self, W2_neigh, b2):
    raise NotImplementedError("write your pallas kernel here")


# speedup vs baseline: 11.2172x; 11.2172x over previous
import functools

import jax
import jax.numpy as jnp
from jax import lax
from jax.experimental import pallas as pl
from jax.experimental.pallas import tpu as pltpu
from jax.experimental.pallas import tpu_sc as plsc

N_NODES = 50000
N_EDGES = 800000
IN_FEATS = 3
H_FEATS = 150
NUM_OUT = 16

NC = 2
NS = 16
L = 16
NW = NC * NS
CH = 128
NCH = 200
EPW = CH * NCH
E_PAD = EPW * NW
N_TILE_ROWS = 3136
N_PAD = N_TILE_ROWS * NS
ZCH = N_TILE_ROWS // 2

ROWS_B = 1000
GRID = N_NODES // ROWS_B


def _edge_pass(table, src, dst):
    mesh = plsc.VectorSubcoreMesh(core_axis_name="c", subcore_axis_name="s")

    @functools.partial(
        pl.kernel,
        mesh=mesh,
        compiler_params=pltpu.CompilerParams(use_tc_tiling_on_sc=False),
        out_type=jax.ShapeDtypeStruct((NC, N_PAD, L), jnp.float32),
        scratch_types=[
            pltpu.VMEM((NCH, CH), jnp.int32),
            pltpu.VMEM((NCH, CH), jnp.int32),
            pltpu.VMEM((CH, L), jnp.float32),
            pltpu.VMEM((ZCH, L), jnp.float32),
            pltpu.VMEM_SHARED((N_PAD, L), jnp.float32),
            pltpu.SemaphoreType.DMA,
        ],
    )
    def k(table_hbm, src_hbm, dst_hbm, out_hbm,
          src_v, dst_v, rows_v, stage_v, acc_sh, sem):
        c = lax.axis_index("c")
        s = lax.axis_index("s")
        w = c * NS + s

        pltpu.sync_copy(src_hbm.at[w], src_v)
        pltpu.sync_copy(dst_hbm.at[w], dst_v)

        def zrow(i, carry):
            stage_v[i, :] = jnp.zeros((L,), jnp.float32)
            return carry

        lax.fori_loop(0, ZCH, zrow, 0)
        base = s * N_TILE_ROWS
        pltpu.sync_copy(stage_v, acc_sh.at[pl.ds(base, ZCH)])
        pltpu.sync_copy(stage_v, acc_sh.at[pl.ds(base + ZCH, ZCH)])
        plsc.subcore_barrier()

        def body(j, carry):
            pltpu.async_copy(table_hbm.at[src_v.at[j]], rows_v, sem).wait()
            pltpu.sync_copy(rows_v, acc_sh.at[dst_v.at[j]], add=True)
            return carry

        lax.fori_loop(0, NCH, body, 0)
        plsc.subcore_barrier()

        pltpu.sync_copy(acc_sh.at[pl.ds(base, ZCH)], stage_v)
        pltpu.sync_copy(stage_v, out_hbm.at[c, pl.ds(base, ZCH)])
        pltpu.sync_copy(acc_sh.at[pl.ds(base + ZCH, ZCH)], stage_v)
        pltpu.sync_copy(stage_v, out_hbm.at[c, pl.ds(base + ZCH, ZCH)])

    return k(table, src, dst)


def _mid_body(p_ref, x_ref, w1s_ref, w1n_ref, b1_ref, w2n_ref,
              h_ref, hw_ref, dinv_ref):
    agg = p_ref[0] + p_ref[1]
    dinv = 1.0 / jnp.maximum(agg[:, 3:4], 1.0)
    hn = agg[:, 0:3] * dinv
    h = (jnp.dot(x_ref[...], w1s_ref[...], preferred_element_type=jnp.float32)
         + jnp.dot(hn, w1n_ref[...], preferred_element_type=jnp.float32)
         + b1_ref[...])
    h = jnp.maximum(h, 0.0)
    h_ref[...] = h
    hw_ref[...] = jnp.dot(h, w2n_ref[...], preferred_element_type=jnp.float32)
    dinv_ref[...] = dinv


def _dense_mid(p, x, w1s, w1n, b1, w2n):
    return pl.pallas_call(
        _mid_body,
        grid=(GRID,),
        in_specs=[
            pl.BlockSpec((NC, ROWS_B, L), lambda i: (0, i, 0)),
            pl.BlockSpec((ROWS_B, IN_FEATS), lambda i: (i, 0)),
            pl.BlockSpec((IN_FEATS, H_FEATS), lambda i: (0, 0)),
            pl.BlockSpec((IN_FEATS, H_FEATS), lambda i: (0, 0)),
            pl.BlockSpec((1, H_FEATS), lambda i: (0, 0)),
            pl.BlockSpec((H_FEATS, NUM_OUT), lambda i: (0, 0)),
        ],
        out_specs=[
            pl.BlockSpec((ROWS_B, H_FEATS), lambda i: (i, 0)),
            pl.BlockSpec((ROWS_B, NUM_OUT), lambda i: (i, 0)),
            pl.BlockSpec((ROWS_B, 1), lambda i: (i, 0)),
        ],
        out_shape=[
            jax.ShapeDtypeStruct((N_NODES, H_FEATS), jnp.float32),
            jax.ShapeDtypeStruct((N_NODES, NUM_OUT), jnp.float32),
            jax.ShapeDtypeStruct((N_NODES, 1), jnp.float32),
        ],
    )(p, x, w1s, w1n, b1, w2n)


def _out_body(h_ref, q_ref, dinv_ref, w2s_ref, b2_ref, o_ref):
    agg2 = q_ref[0] + q_ref[1]
    o_ref[...] = (
        jnp.dot(h_ref[...], w2s_ref[...], preferred_element_type=jnp.float32)
        + agg2 * dinv_ref[...]
        + b2_ref[...])


def _dense_out(h, q, dinv, w2s, b2):
    return pl.pallas_call(
        _out_body,
        grid=(GRID,),
        in_specs=[
            pl.BlockSpec((ROWS_B, H_FEATS), lambda i: (i, 0)),
            pl.BlockSpec((NC, ROWS_B, L), lambda i: (0, i, 0)),
            pl.BlockSpec((ROWS_B, 1), lambda i: (i, 0)),
            pl.BlockSpec((H_FEATS, NUM_OUT), lambda i: (0, 0)),
            pl.BlockSpec((1, NUM_OUT), lambda i: (0, 0)),
        ],
        out_specs=pl.BlockSpec((ROWS_B, NUM_OUT), lambda i: (i, 0)),
        out_shape=jax.ShapeDtypeStruct((N_NODES, NUM_OUT), jnp.float32),
    )(h, q, dinv, w2s, b2)


def kernel(in_feat, edge_index, W1_self, W1_neigh, b1, W2_self, W2_neigh, b2):
    src = edge_index[0].astype(jnp.int32)
    dst = edge_index[1].astype(jnp.int32)
    pad = E_PAD - N_EDGES
    src_p = jnp.concatenate(
        [src, jnp.zeros((pad,), jnp.int32)]).reshape(NW, NCH, CH)
    dst_p = jnp.concatenate(
        [dst, jnp.full((pad,), N_NODES, jnp.int32)]).reshape(NW, NCH, CH)

    table1 = jnp.concatenate(
        [in_feat,
         jnp.ones((N_NODES, 1), jnp.float32),
         jnp.zeros((N_NODES, L - IN_FEATS - 1), jnp.float32)], axis=1)

    p1 = _edge_pass(table1, src_p, dst_p)[:, :N_NODES, :]
    h, hw, dinv = _dense_mid(p1, in_feat, W1_self, W1_neigh,
                             b1.reshape(1, -1), W2_neigh)
    p2 = _edge_pass(hw, src_p, dst_p)[:, :N_NODES, :]
    return _dense_out(h, p2, dinv, W2_self, b2.reshape(1, -1))
